# baseline (speedup 1.0000x reference)
import functools

import numpy as np
import jax
import jax.numpy as jnp
from jax import lax
from jax.experimental import pallas as pl
from jax.experimental.pallas import tpu as pltpu
from jax.experimental.pallas import tpu_sc as plsc

N_LEVELS = 16
F = 2
LOG2_T = 19
T = 2 ** LOG2_T
HASH_MASK = T - 1
BASE_RES = 16.0
FINEST_RES = 512.0
N_PTS = 262144
B_GROWTH = float(np.exp((np.log(FINEST_RES) - np.log(BASE_RES)) / (N_LEVELS - 1)))
_RES = [float(np.floor(np.float32(BASE_RES * (B_GROWTH ** i)))) for i in range(N_LEVELS)]
_GS = [float(np.float32(2.0) / np.float32(r)) for r in _RES]
P1_I = 2654435761 - 2 ** 32
P2_I = 805459861

NC = 2
NS = 16
NW = NC * NS
PW = N_PTS // NW
C = 512
NCHUNK = PW // C
GROUPS = C // 16
NIDX = 8 * C

N_COMPACT = 4
_R1 = [int(_RES[i]) + 1 for i in range(N_COMPACT)]
_CENT = [r1 ** 3 for r1 in _R1]
_CPAD = [-(-e // 16) * 16 for e in _CENT]
SEG = NIDX


def _sc_body(xt_hbm, tbl_hbm, out_hbm, x_v, w0_v, w1_v, idx0_v, idx1_v,
             rows0_v, rows1_v, out_v, ct0_v, ct1_v, ct2_v, ct3_v, sem0, sem1):
    c_id = lax.axis_index("c")
    s_id = lax.axis_index("s")
    wid = s_id * NC + c_id
    iota = lax.iota(jnp.int32, 16)
    compact_refs = [ct0_v, ct1_v, ct2_v, ct3_v]

    for lvl in range(N_COMPACT):
        r1 = _R1[lvl]
        r1sq = r1 * r1
        lvl_off = jnp.int32(lvl * T)
        cref = compact_refs[lvl]
        for seg in range(-(-_CPAD[lvl] // SEG)):
            seglen = min(SEG, _CPAD[lvl] - seg * SEG)

            def fill(g, carry, seg=seg, r1=r1, r1sq=r1sq, lvl_off=lvl_off):
                n = jnp.int32(seg * SEG) + g * jnp.int32(16) + iota
                a = n // jnp.int32(r1sq)
                rem = n - a * jnp.int32(r1sq)
                b = rem // jnp.int32(r1)
                cc = rem - b * jnp.int32(r1)
                h = a ^ (b * jnp.int32(P1_I)) ^ (cc * jnp.int32(P2_I))
                idx0_v[pl.ds(g * 16, 16)] = (h & jnp.int32(HASH_MASK)) + lvl_off
                return carry

            lax.fori_loop(0, seglen // 16, fill, 0)
            pltpu.async_copy(
                tbl_hbm.at[idx0_v.at[pl.ds(0, seglen)]],
                cref.at[pl.ds(seg * SEG, seglen)], sem0).wait()

    def unpack_interp(pw, wx, wy, wz, lvl, p0):
        ef = [[plsc.bitcast(w << jnp.int32(16), jnp.float32) for w in pw],
              [plsc.bitcast(w & jnp.int32(-65536), jnp.float32) for w in pw]]
        for f in range(F):
            e = ef[f]
            c00 = e[0] + wx * (e[4] - e[0])
            c01 = e[1] + wx * (e[5] - e[1])
            c10 = e[2] + wx * (e[6] - e[2])
            c11 = e[3] + wx * (e[7] - e[3])
            c0 = c00 + wy * (c10 - c00)
            c1 = c01 + wy * (c11 - c01)
            out_v[2 * lvl + f, pl.ds(p0, 16)] = c0 + wz * (c1 - c0)

    def coords(lvl, p0):
        gs = jnp.float32(_GS[lvl])
        res = _RES[lvl]
        pow2 = (res == 2.0 ** int(np.log2(res)))
        inv_gs = float(np.float32(1.0) / np.float32(_GS[lvl]))
        bl, w = [], []
        for d in range(3):
            xd = x_v[d, pl.ds(p0, 16)]
            xs = xd - jnp.float32(-1.0)
            s = xs * jnp.float32(res / 2.0) if pow2 else xs / gs
            bli = s.astype(jnp.int32)
            blf = bli.astype(jnp.float32)
            vmin = blf * gs + jnp.float32(-1.0)
            w.append((xd - vmin) * jnp.float32(inv_gs))
            bl.append(bli)
        return bl, w

    def make_compact_level(lvl):
        r1 = _R1[lvl]
        r1sq = r1 * r1
        cref = compact_refs[lvl]

        def body(g, carry):
            p0 = pl.multiple_of(g * 16, 16)
            bl, w = coords(lvl, p0)
            cidx = (bl[0] * jnp.int32(r1) + bl[1]) * jnp.int32(r1) + bl[2]
            pw = [plsc.load_gather(cref, [cidx + jnp.int32(i * r1sq + j * r1 + k)])
                  for i in (0, 1) for j in (0, 1) for k in (0, 1)]
            unpack_interp(pw, w[0], w[1], w[2], lvl, p0)
            return carry

        lax.fori_loop(0, GROUPS, body, 0)

    def make_phase_a(lvl, idx_ref, w_ref):
        lvl_off = jnp.int32(lvl * T)

        def phase_a(g, carry):
            p0 = pl.multiple_of(g * 16, 16)
            bl, w = coords(lvl, p0)
            for d in range(3):
                w_ref[d, pl.ds(p0, 16)] = w[d]
            m0 = bl[0]
            m0b = m0 + jnp.int32(1)
            m1 = bl[1] * jnp.int32(P1_I)
            m1b = m1 + jnp.int32(P1_I)
            m2 = bl[2] * jnp.int32(P2_I)
            m2b = m2 + jnp.int32(P2_I)
            e00 = m0 ^ m1
            e01 = m0 ^ m1b
            e10 = m0b ^ m1
            e11 = m0b ^ m1b
            corners = (e00 ^ m2, e00 ^ m2b, e01 ^ m2, e01 ^ m2b,
                       e10 ^ m2, e10 ^ m2b, e11 ^ m2, e11 ^ m2b)
            for c, h in enumerate(corners):
                idx_ref[pl.ds(c * C + p0, 16)] = (h & jnp.int32(HASH_MASK)) + lvl_off
            return carry

        lax.fori_loop(0, GROUPS, phase_a, 0)

    def make_phase_b(lvl, rows_ref, w_ref):
        def phase_b(g, carry):
            p0 = pl.multiple_of(g * 16, 16)
            wx = w_ref[0, pl.ds(p0, 16)]
            wy = w_ref[1, pl.ds(p0, 16)]
            wz = w_ref[2, pl.ds(p0, 16)]
            pw = [rows_ref[pl.ds(c * C + p0, 16)] for c in range(8)]
            unpack_interp(pw, wx, wy, wz, lvl, p0)
            return carry

        lax.fori_loop(0, GROUPS, phase_b, 0)

    bufs = [(idx0_v, rows0_v, w0_v, sem0), (idx1_v, rows1_v, w1_v, sem1)]

    def chunk_body(ch, carry):
        base = (wid * NCHUNK + ch) * C
        pltpu.sync_copy(xt_hbm.at[:, pl.ds(base, C)], x_v)

        cps = {}

        def start(lvl):
            idx_r, rows_r, w_r, sem_r = bufs[lvl % 2]
            make_phase_a(lvl, idx_r, w_r)
            cps[lvl] = pltpu.async_copy(tbl_hbm.at[idx_r], rows_r, sem_r)

        start(N_COMPACT)
        start(N_COMPACT + 1)
        for lvl in range(N_COMPACT):
            make_compact_level(lvl)
        for lvl in range(N_COMPACT, N_LEVELS):
            cps[lvl].wait()
            _, rows_l, w_l, _ = bufs[lvl % 2]
            make_phase_b(lvl, rows_l, w_l)
            if lvl + 2 < N_LEVELS:
                start(lvl + 2)
        pltpu.sync_copy(out_v, out_hbm.at[:, pl.ds(base, C)])
        return carry

    lax.fori_loop(0, NCHUNK, chunk_body, 0)


@functools.lru_cache(maxsize=1)
def _make_sc_call():
    mesh = plsc.VectorSubcoreMesh(
        core_axis_name="c", subcore_axis_name="s", num_cores=NC, num_subcores=NS
    )
    return pl.kernel(
        _sc_body,
        out_type=jax.ShapeDtypeStruct((2 * N_LEVELS, N_PTS), jnp.float32),
        mesh=mesh,
        compiler_params=pltpu.CompilerParams(needs_layout_passes=False),
        scratch_types=[
            pltpu.VMEM((3, C), jnp.float32),
            pltpu.VMEM((3, C), jnp.float32),
            pltpu.VMEM((3, C), jnp.float32),
            pltpu.VMEM((NIDX,), jnp.int32),
            pltpu.VMEM((NIDX,), jnp.int32),
            pltpu.VMEM((NIDX,), jnp.int32),
            pltpu.VMEM((NIDX,), jnp.int32),
            pltpu.VMEM((2 * N_LEVELS, C), jnp.float32),
            pltpu.VMEM((_CPAD[0],), jnp.int32),
            pltpu.VMEM((_CPAD[1],), jnp.int32),
            pltpu.VMEM((_CPAD[2],), jnp.int32),
            pltpu.VMEM((_CPAD[3],), jnp.int32),
            pltpu.SemaphoreType.DMA,
            pltpu.SemaphoreType.DMA,
        ],
    )


@jax.jit
def kernel(x, tables):
    xt = x.T
    b = jax.lax.bitcast_convert_type(tables.astype(jnp.bfloat16), jnp.uint16)
    packed = (b[..., 0].astype(jnp.uint32)
              | (b[..., 1].astype(jnp.uint32) << 16)).astype(jnp.int32)
    tbl = packed.reshape(N_LEVELS * T)
    out = _make_sc_call()(xt, tbl)
    return out.T


# speedup vs baseline: 1.0029x; 1.0029x over previous
import functools

import numpy as np
import jax
import jax.numpy as jnp
from jax import lax
from jax.experimental import pallas as pl
from jax.experimental.pallas import tpu as pltpu
from jax.experimental.pallas import tpu_sc as plsc

N_LEVELS = 16
F = 2
LOG2_T = 19
T = 2 ** LOG2_T
HASH_MASK = T - 1
BASE_RES = 16.0
FINEST_RES = 512.0
N_PTS = 262144
B_GROWTH = float(np.exp((np.log(FINEST_RES) - np.log(BASE_RES)) / (N_LEVELS - 1)))
_RES = [float(np.floor(np.float32(BASE_RES * (B_GROWTH ** i)))) for i in range(N_LEVELS)]
_GS = [float(np.float32(2.0) / np.float32(r)) for r in _RES]
P1_I = 2654435761 - 2 ** 32
P2_I = 805459861

NC = 2
NS = 16
NW = NC * NS
PW = N_PTS // NW
C = 512
NCHUNK = PW // C
GROUPS = C // 16
NIDX = 8 * C

N_COMPACT = 4
_R1 = [int(_RES[i]) + 1 for i in range(N_COMPACT)]
_CENT = [r1 ** 3 for r1 in _R1]
_CPAD = [-(-e // 16) * 16 for e in _CENT]
SEG = NIDX


def _sc_body(xt_hbm, tbl_hbm, out_hbm, x_v, w0_v, w1_v, w2_v, idx0_v, idx1_v,
             idx2_v, rows0_v, rows1_v, rows2_v, out_v, ct0_v, ct1_v, ct2_v,
             ct3_v, sem0, sem1, sem2):
    c_id = lax.axis_index("c")
    s_id = lax.axis_index("s")
    wid = s_id * NC + c_id
    iota = lax.iota(jnp.int32, 16)
    compact_refs = [ct0_v, ct1_v, ct2_v, ct3_v]

    for lvl in range(N_COMPACT):
        r1 = _R1[lvl]
        r1sq = r1 * r1
        lvl_off = jnp.int32(lvl * T)
        cref = compact_refs[lvl]
        for seg in range(-(-_CPAD[lvl] // SEG)):
            seglen = min(SEG, _CPAD[lvl] - seg * SEG)

            def fill(g, carry, seg=seg, r1=r1, r1sq=r1sq, lvl_off=lvl_off):
                n = jnp.int32(seg * SEG) + g * jnp.int32(16) + iota
                a = n // jnp.int32(r1sq)
                rem = n - a * jnp.int32(r1sq)
                b = rem // jnp.int32(r1)
                cc = rem - b * jnp.int32(r1)
                h = a ^ (b * jnp.int32(P1_I)) ^ (cc * jnp.int32(P2_I))
                idx0_v[pl.ds(g * 16, 16)] = (h & jnp.int32(HASH_MASK)) + lvl_off
                return carry

            lax.fori_loop(0, seglen // 16, fill, 0)
            pltpu.async_copy(
                tbl_hbm.at[idx0_v.at[pl.ds(0, seglen)]],
                cref.at[pl.ds(seg * SEG, seglen)], sem0).wait()

    def unpack_interp(pw, wx, wy, wz, lvl, p0):
        ef = [[plsc.bitcast(w << jnp.int32(16), jnp.float32) for w in pw],
              [plsc.bitcast(w & jnp.int32(-65536), jnp.float32) for w in pw]]
        for f in range(F):
            e = ef[f]
            c00 = e[0] + wx * (e[4] - e[0])
            c01 = e[1] + wx * (e[5] - e[1])
            c10 = e[2] + wx * (e[6] - e[2])
            c11 = e[3] + wx * (e[7] - e[3])
            c0 = c00 + wy * (c10 - c00)
            c1 = c01 + wy * (c11 - c01)
            out_v[2 * lvl + f, pl.ds(p0, 16)] = c0 + wz * (c1 - c0)

    def coords(lvl, p0):
        gs = jnp.float32(_GS[lvl])
        res = _RES[lvl]
        pow2 = (res == 2.0 ** int(np.log2(res)))
        inv_gs = float(np.float32(1.0) / np.float32(_GS[lvl]))
        bl, w = [], []
        for d in range(3):
            xd = x_v[d, pl.ds(p0, 16)]
            xs = xd - jnp.float32(-1.0)
            s = xs * jnp.float32(res / 2.0) if pow2 else xs / gs
            bli = s.astype(jnp.int32)
            blf = bli.astype(jnp.float32)
            vmin = blf * gs + jnp.float32(-1.0)
            w.append((xd - vmin) * jnp.float32(inv_gs))
            bl.append(bli)
        return bl, w

    def make_compact_level(lvl):
        r1 = _R1[lvl]
        r1sq = r1 * r1
        cref = compact_refs[lvl]

        def body(g, carry):
            p0 = pl.multiple_of(g * 16, 16)
            bl, w = coords(lvl, p0)
            cidx = (bl[0] * jnp.int32(r1) + bl[1]) * jnp.int32(r1) + bl[2]
            pw = [plsc.load_gather(cref, [cidx + jnp.int32(i * r1sq + j * r1 + k)])
                  for i in (0, 1) for j in (0, 1) for k in (0, 1)]
            unpack_interp(pw, w[0], w[1], w[2], lvl, p0)
            return carry

        lax.fori_loop(0, GROUPS, body, 0)

    def make_phase_a(lvl, idx_ref, w_ref):
        lvl_off = jnp.int32(lvl * T)

        def phase_a(g, carry):
            p0 = pl.multiple_of(g * 16, 16)
            bl, w = coords(lvl, p0)
            for d in range(3):
                w_ref[d, pl.ds(p0, 16)] = w[d]
            m0 = bl[0]
            m0b = m0 + jnp.int32(1)
            m1 = bl[1] * jnp.int32(P1_I)
            m1b = m1 + jnp.int32(P1_I)
            m2 = bl[2] * jnp.int32(P2_I)
            m2b = m2 + jnp.int32(P2_I)
            e00 = m0 ^ m1
            e01 = m0 ^ m1b
            e10 = m0b ^ m1
            e11 = m0b ^ m1b
            corners = (e00 ^ m2, e00 ^ m2b, e01 ^ m2, e01 ^ m2b,
                       e10 ^ m2, e10 ^ m2b, e11 ^ m2, e11 ^ m2b)
            for c, h in enumerate(corners):
                idx_ref[pl.ds(c * C + p0, 16)] = (h & jnp.int32(HASH_MASK)) + lvl_off
            return carry

        lax.fori_loop(0, GROUPS, phase_a, 0)

    def make_phase_b(lvl, rows_ref, w_ref):
        def phase_b(g, carry):
            p0 = pl.multiple_of(g * 16, 16)
            wx = w_ref[0, pl.ds(p0, 16)]
            wy = w_ref[1, pl.ds(p0, 16)]
            wz = w_ref[2, pl.ds(p0, 16)]
            pw = [rows_ref[pl.ds(c * C + p0, 16)] for c in range(8)]
            unpack_interp(pw, wx, wy, wz, lvl, p0)
            return carry

        lax.fori_loop(0, GROUPS, phase_b, 0)

    bufs = [(idx0_v, rows0_v, w0_v, sem0), (idx1_v, rows1_v, w1_v, sem1),
            (idx2_v, rows2_v, w2_v, sem2)]

    def chunk_body(ch, carry):
        base = (wid * NCHUNK + ch) * C
        pltpu.sync_copy(xt_hbm.at[:, pl.ds(base, C)], x_v)

        cps = {}

        def start(lvl):
            idx_r, rows_r, w_r, sem_r = bufs[lvl % 3]
            make_phase_a(lvl, idx_r, w_r)
            cps[lvl] = pltpu.async_copy(tbl_hbm.at[idx_r], rows_r, sem_r)

        start(N_COMPACT)
        start(N_COMPACT + 1)
        start(N_COMPACT + 2)
        for lvl in range(N_COMPACT):
            make_compact_level(lvl)
        for lvl in range(N_COMPACT, N_LEVELS):
            cps[lvl].wait()
            _, rows_l, w_l, _ = bufs[lvl % 3]
            make_phase_b(lvl, rows_l, w_l)
            if lvl + 3 < N_LEVELS:
                start(lvl + 3)
        pltpu.sync_copy(out_v, out_hbm.at[:, pl.ds(base, C)])
        return carry

    lax.fori_loop(0, NCHUNK, chunk_body, 0)


@functools.lru_cache(maxsize=1)
def _make_sc_call():
    mesh = plsc.VectorSubcoreMesh(
        core_axis_name="c", subcore_axis_name="s", num_cores=NC, num_subcores=NS
    )
    return pl.kernel(
        _sc_body,
        out_type=jax.ShapeDtypeStruct((2 * N_LEVELS, N_PTS), jnp.float32),
        mesh=mesh,
        compiler_params=pltpu.CompilerParams(needs_layout_passes=False),
        scratch_types=[
            pltpu.VMEM((3, C), jnp.float32),
            pltpu.VMEM((3, C), jnp.float32),
            pltpu.VMEM((3, C), jnp.float32),
            pltpu.VMEM((3, C), jnp.float32),
            pltpu.VMEM((NIDX,), jnp.int32),
            pltpu.VMEM((NIDX,), jnp.int32),
            pltpu.VMEM((NIDX,), jnp.int32),
            pltpu.VMEM((NIDX,), jnp.int32),
            pltpu.VMEM((NIDX,), jnp.int32),
            pltpu.VMEM((NIDX,), jnp.int32),
            pltpu.VMEM((2 * N_LEVELS, C), jnp.float32),
            pltpu.VMEM((_CPAD[0],), jnp.int32),
            pltpu.VMEM((_CPAD[1],), jnp.int32),
            pltpu.VMEM((_CPAD[2],), jnp.int32),
            pltpu.VMEM((_CPAD[3],), jnp.int32),
            pltpu.SemaphoreType.DMA,
            pltpu.SemaphoreType.DMA,
            pltpu.SemaphoreType.DMA,
        ],
    )


@jax.jit
def kernel(x, tables):
    xt = x.T
    b = jax.lax.bitcast_convert_type(tables.astype(jnp.bfloat16), jnp.uint16)
    packed = (b[..., 0].astype(jnp.uint32)
              | (b[..., 1].astype(jnp.uint32) << 16)).astype(jnp.int32)
    tbl = packed.reshape(N_LEVELS * T)
    out = _make_sc_call()(xt, tbl)
    return out.T
